# baseline (speedup 1.0000x reference)
import functools

import jax
import jax.numpy as jnp
from jax import lax
from jax.experimental import pallas as pl
from jax.experimental.pallas import tpu as pltpu
from jax.experimental.pallas import tpu_sc as plsc

VOCAB = 1000000
EMB = 32
B = 4096
L = 200
N = B * L
NC = 2
NS = 16
NW = NC * NS
PER_W = N // NW
SENT_W = B // NW
NBUF = 4
ROUNDS = L // NBUF
VBLK = 8192
VGRID = -(-VOCAB // VBLK)
VPAD = VGRID * VBLK

_mesh = plsc.VectorSubcoreMesh(core_axis_name="c", subcore_axis_name="s")


def _transpose_body(wt_ref, out_ref):
    xt = jnp.transpose(wt_ref[...])
    for q in range(4):
        out_ref[:, q * EMB:(q + 1) * EMB] = xt[q * (VBLK // 4):(q + 1) * (VBLK // 4), :]


_transpose = pl.pallas_call(
    _transpose_body,
    grid=(VGRID,),
    in_specs=[pl.BlockSpec((EMB, VBLK), lambda i: (0, i))],
    out_specs=pl.BlockSpec((VBLK // 4, 128), lambda i: (i, 0)),
    out_shape=jax.ShapeDtypeStruct((VPAD * EMB // 128, 128), jnp.float32),
)


@functools.partial(
    pl.kernel,
    mesh=_mesh,
    out_type=jax.ShapeDtypeStruct((L, EMB, B), jnp.float32),
    compiler_params=pltpu.CompilerParams(
        use_tc_tiling_on_sc=False, needs_layout_passes=False
    ),
    scratch_types=[
        pltpu.VMEM((PER_W + 16,), jnp.int32),
        pltpu.VMEM((PER_W,), jnp.int32),
        [pltpu.VMEM((SENT_W, EMB), jnp.float32) for _ in range(NBUF)],
        [pltpu.VMEM((1, EMB, SENT_W), jnp.float32) for _ in range(NBUF)],
        [pltpu.SemaphoreType.DMA for _ in range(NBUF)],
        [pltpu.SemaphoreType.DMA for _ in range(NBUF)],
    ],
)
def _gather_kernel(idx_hbm, table_hbm, out_hbm, idx_v, idx_t, rows, tbuf,
                   sem_g, sem_s):
    wid = lax.axis_index("s") * NC + lax.axis_index("c")
    base = wid * PER_W
    bbase = wid * SENT_W

    pltpu.sync_copy(idx_hbm.at[pl.ds(base, PER_W)], idx_v.at[pl.ds(0, PER_W)])

    iota = lax.iota(jnp.int32, 16)
    tail_mask = iota < (L - (L // 16) * 16)

    def reorder(j, carry):
        for c in range(L // 16 + 1):
            vec = idx_v[pl.ds(j * L + c * 16, 16)]
            dst = (c * 16 + iota) * SENT_W + j
            if c < L // 16:
                plsc.store_scatter(idx_t, [dst], vec)
            else:
                plsc.store_scatter(idx_t, [dst], vec, mask=tail_mask)
        return carry

    lax.fori_loop(0, SENT_W, reorder, 0)

    def fire(l, b):
        return pltpu.async_copy(
            table_hbm.at[idx_t.at[pl.ds(l * SENT_W, SENT_W)]],
            rows[b],
            sem_g[b],
        )

    def store(l, b):
        return pltpu.make_async_copy(
            tbuf[b],
            out_hbm.at[pl.ds(l, 1), pl.ds(0, EMB), pl.ds(bbase, SENT_W)],
            sem_s[b],
        )

    def transpose(b):
        for e in range(EMB):
            esplat = jnp.full((16,), e, jnp.int32)
            for g in range(SENT_W // 16):
                vec = plsc.load_gather(rows[b], [g * 16 + iota, esplat])
                tbuf[b][0, e, pl.ds(g * 16, 16)] = vec

    def body(r, carry):
        l0 = r * NBUF
        descs = []
        for b in range(NBUF):
            @pl.when(r > 0)
            def _():
                store(0, b).wait()
            descs.append(fire(l0 + b, b))
        for b in range(NBUF):
            descs[b].wait()
            transpose(b)
            store(l0 + b, b).start()
        return carry

    lax.fori_loop(0, ROUNDS, body, 0)

    for b in range(NBUF):
        store(0, b).wait()


def kernel(sent_words, embed_weight):
    idx = sent_words.reshape(-1).astype(jnp.int32)
    rem = idx % VBLK
    idxp = (idx - rem) + 4 * (rem % (VBLK // 4)) + rem // (VBLK // 4)
    table_rm = _transpose(embed_weight.T).reshape(VPAD, EMB)
    out = _gather_kernel(idxp, table_rm)
    return jnp.transpose(out, (2, 0, 1))


# speedup vs baseline: 1.6993x; 1.6993x over previous
import functools

import jax
import jax.numpy as jnp
from jax import lax
from jax.experimental import pallas as pl
from jax.experimental.pallas import tpu as pltpu
from jax.experimental.pallas import tpu_sc as plsc

VOCAB = 1000000
EMB = 32
B = 4096
L = 200
N = B * L
NC = 2
NS = 16
NW = NC * NS
PER_W = N // NW
SENT_W = B // NW
NBUF = 2
ROUNDS = L // NBUF
TSTRIDE = 129
VBLK = 8192
VGRID = -(-VOCAB // VBLK)
VPAD = VGRID * VBLK

_mesh = plsc.VectorSubcoreMesh(core_axis_name="c", subcore_axis_name="s")


def _transpose_body(wt_ref, out_ref):
    xt = jnp.transpose(wt_ref[...])
    for q in range(4):
        out_ref[:, q * EMB:(q + 1) * EMB] = xt[q * (VBLK // 4):(q + 1) * (VBLK // 4), :]


_transpose = pl.pallas_call(
    _transpose_body,
    grid=(VGRID,),
    in_specs=[pl.BlockSpec((EMB, VBLK), lambda i: (0, i))],
    out_specs=pl.BlockSpec((VBLK // 4, 128), lambda i: (i, 0)),
    out_shape=jax.ShapeDtypeStruct((VPAD * EMB // 128, 128), jnp.float32),
)


@functools.partial(
    pl.kernel,
    mesh=_mesh,
    out_type=jax.ShapeDtypeStruct((L, EMB, B), jnp.float32),
    compiler_params=pltpu.CompilerParams(
        use_tc_tiling_on_sc=False, needs_layout_passes=False
    ),
    scratch_types=[
        pltpu.VMEM((PER_W,), jnp.int32),
        [pltpu.VMEM((SENT_W, EMB), jnp.float32) for _ in range(NBUF)],
        [pltpu.VMEM((1, EMB, TSTRIDE), jnp.float32) for _ in range(NBUF)],
        [pltpu.SemaphoreType.DMA for _ in range(NBUF)],
        [pltpu.SemaphoreType.DMA for _ in range(NBUF)],
    ],
)
def _gather_kernel(idx_hbm, table_hbm, out_hbm, idx_v, rows, tbuf,
                   sem_g, sem_s):
    wid = lax.axis_index("s") * NC + lax.axis_index("c")
    base = wid * PER_W
    bbase = wid * SENT_W

    pltpu.sync_copy(idx_hbm.at[pl.ds(base, PER_W)], idx_v)

    iota = lax.iota(jnp.int32, 16)
    zeros16 = jnp.zeros((16,), jnp.int32)
    echunk = [c * 16 + iota for c in range(EMB // 16)]

    def fire(l, b):
        return pltpu.async_copy(
            table_hbm.at[idx_v.at[pl.ds(l * SENT_W, SENT_W)]],
            rows[b],
            sem_g[b],
        )

    def store(l, b):
        return pltpu.make_async_copy(
            tbuf[b].at[:, :, pl.ds(0, SENT_W)],
            out_hbm.at[pl.ds(l, 1), pl.ds(0, EMB), pl.ds(bbase, SENT_W)],
            sem_s[b],
        )

    def transpose(b):
        for r in range(SENT_W):
            rsplat = jnp.full((16,), r, jnp.int32)
            for c in range(EMB // 16):
                vec = rows[b][r, pl.ds(c * 16, 16)]
                plsc.store_scatter(tbuf[b], [zeros16, echunk[c], rsplat], vec)

    def body(r, carry):
        l0 = r * NBUF
        descs = []
        for b in range(NBUF):
            @pl.when(r > 0)
            def _():
                store(0, b).wait()
            descs.append(fire(l0 + b, b))
        for b in range(NBUF):
            descs[b].wait()
            transpose(b)
            store(l0 + b, b).start()
        return carry

    lax.fori_loop(0, ROUNDS, body, 0)

    for b in range(NBUF):
        store(0, b).wait()


def kernel(sent_words, embed_weight):
    idx = sent_words.reshape(-1).astype(jnp.int32)
    rem = idx % VBLK
    idxp = (idx - rem) + 4 * (rem % (VBLK // 4)) + rem // (VBLK // 4)
    idxt = idxp.reshape(NW, SENT_W, L).transpose(0, 2, 1).reshape(-1)
    table_rm = _transpose(embed_weight.T).reshape(VPAD, EMB)
    out = _gather_kernel(idxt, table_rm)
    return jnp.transpose(out, (2, 0, 1))
